# Initial kernel scaffold; baseline (speedup 1.0000x reference)
import jax
import jax.numpy as jnp
from jax.experimental import pallas as pl


def kernel(x, net, bn_gamma, bn_beta, w_in, b_in, gcn_weights, w_cls, b_cls):
    raise NotImplementedError("write your pallas kernel here")


# speedup vs baseline: 1.3625x; 1.3625x over previous
import functools
import math

import jax
import jax.numpy as jnp
from jax.experimental import pallas as pl
from jax.experimental.pallas import tpu as pltpu

ALPHA = 0.5
LAMDA = 0.5
NLAYERS = 8


def _prelude(x_ref, g_ref, b_ref, w_ref, bi_ref, x0_ref, h0_ref):
    x = x_ref[...]
    mean = jnp.mean(x, axis=0, keepdims=True)
    var = jnp.mean((x - mean) ** 2, axis=0, keepdims=True)
    xn = (x - mean) / jnp.sqrt(var + 1e-5) * g_ref[...] + b_ref[...]
    x0 = jax.nn.relu(
        jnp.dot(xn, w_ref[...], preferred_element_type=jnp.float32) + bi_ref[...]
    )
    x0_ref[...] = x0
    h0_ref[...] = x0.astype(jnp.bfloat16)


def _gcn_core(net_ref, h_ref, x0_ref, w_ref, beta):
    hi = jnp.dot(
        net_ref[...].astype(jnp.bfloat16), h_ref[...],
        preferred_element_type=jnp.float32,
    )
    support = (1.0 - ALPHA) * hi + ALPHA * x0_ref[...]
    sw = jnp.dot(support, w_ref[...], preferred_element_type=jnp.float32)
    return jax.nn.relu(beta * sw + (1.0 - beta) * support)


def _layer_first(net_ref, h_ref, x0_ref, w_ref, hn_ref, netlp_ref, *, beta):
    hn_ref[...] = _gcn_core(net_ref, h_ref, x0_ref, w_ref, beta).astype(jnp.bfloat16)
    netlp_ref[...] = net_ref[...].astype(jnp.bfloat16)


def _layer_mid(net_ref, h_ref, x0_ref, w_ref, hn_ref, *, beta):
    hn_ref[...] = _gcn_core(net_ref, h_ref, x0_ref, w_ref, beta).astype(jnp.bfloat16)


def _layer_last(net_ref, h_ref, x0_ref, w_ref, wc_ref, bc_ref, pred_ref, *, beta):
    h = _gcn_core(net_ref, h_ref, x0_ref, w_ref, beta)
    pred_ref[...] = (
        jnp.dot(h, wc_ref[...], preferred_element_type=jnp.float32) + bc_ref[...]
    )


def kernel(x, net, bn_gamma, bn_beta, w_in, b_in, gcn_weights, w_cls, b_cls):
    n, nfeat = x.shape
    dim = w_in.shape[1]
    nclass = w_cls.shape[1]
    br = 400 if n % 400 == 0 else n
    nblk = n // br

    g2 = bn_gamma.reshape(1, nfeat)
    b2 = bn_beta.reshape(1, nfeat)
    bi2 = b_in.reshape(1, dim)
    bc2 = b_cls.reshape(1, nclass)

    x0, h = pl.pallas_call(
        _prelude,
        out_shape=[
            jax.ShapeDtypeStruct((n, dim), jnp.float32),
            jax.ShapeDtypeStruct((n, dim), jnp.bfloat16),
        ],
    )(x, g2, b2, w_in, bi2)

    full = lambda i: (0, 0)
    rows = lambda i: (i, 0)
    h_spec = pl.BlockSpec((n, dim), full)
    x0_spec = pl.BlockSpec((br, dim), rows)
    w_spec = pl.BlockSpec((dim, dim), full)
    hn_spec = pl.BlockSpec((br, dim), rows)
    hn_shape = jax.ShapeDtypeStruct((n, dim), jnp.bfloat16)
    params = pltpu.CompilerParams(dimension_semantics=("parallel",))

    net_lp = None
    for l in range(NLAYERS):
        beta = math.log(LAMDA / (l + 1) + 1.0)
        wl = gcn_weights[l]
        if l == 0:
            h, net_lp = pl.pallas_call(
                functools.partial(_layer_first, beta=beta),
                grid=(nblk,),
                in_specs=[pl.BlockSpec((br, n), rows), h_spec, x0_spec, w_spec],
                out_specs=[hn_spec, pl.BlockSpec((br, n), rows)],
                out_shape=[hn_shape, jax.ShapeDtypeStruct((n, n), jnp.bfloat16)],
                compiler_params=params,
            )(net, h, x0, wl)
        elif l < NLAYERS - 1:
            h = pl.pallas_call(
                functools.partial(_layer_mid, beta=beta),
                grid=(nblk,),
                in_specs=[pl.BlockSpec((br, n), rows), h_spec, x0_spec, w_spec],
                out_specs=hn_spec,
                out_shape=hn_shape,
                compiler_params=params,
            )(net_lp, h, x0, wl)
        else:
            pred = pl.pallas_call(
                functools.partial(_layer_last, beta=beta),
                grid=(nblk,),
                in_specs=[
                    pl.BlockSpec((br, n), rows), h_spec, x0_spec, w_spec,
                    pl.BlockSpec((dim, nclass), full),
                    pl.BlockSpec((1, nclass), full),
                ],
                out_specs=pl.BlockSpec((br, nclass), rows),
                out_shape=jax.ShapeDtypeStruct((n, nclass), jnp.float32),
                compiler_params=params,
            )(net_lp, h, x0, wl, w_cls, bc2)
    return pred


# speedup vs baseline: 1.9581x; 1.4372x over previous
import functools
import math

import jax
import jax.numpy as jnp
from jax.experimental import pallas as pl
from jax.experimental.pallas import tpu as pltpu

ALPHA = 0.5
LAMDA = 0.5
NLAYERS = 8
FP8 = jnp.float8_e4m3fn


def _prelude(x_ref, g_ref, b_ref, w_ref, bi_ref, x0_ref, h0_ref):
    x = x_ref[...]
    mean = jnp.mean(x, axis=0, keepdims=True)
    var = jnp.mean((x - mean) ** 2, axis=0, keepdims=True)
    xn = (x - mean) / jnp.sqrt(var + 1e-5) * g_ref[...] + b_ref[...]
    x0 = jax.nn.relu(
        jnp.dot(xn, w_ref[...], preferred_element_type=jnp.float32) + bi_ref[...]
    )
    x0_ref[...] = x0
    h0_ref[...] = x0.astype(jnp.bfloat16)


def _to_fp8(h):
    return jnp.minimum(h, 448.0).astype(FP8)


def _gcn_core(net_ref, h_ref, x0_ref, w_ref, beta, descale):
    if net_ref.dtype == jnp.float32:
        hi = jnp.dot(
            net_ref[...].astype(jnp.bfloat16), h_ref[...],
            preferred_element_type=jnp.float32,
        )
    else:
        hi = jnp.dot(net_ref[...], h_ref[...], preferred_element_type=jnp.float32)
    support = (1.0 - ALPHA) * descale * hi + ALPHA * x0_ref[...]
    sw = jnp.dot(support, w_ref[...], preferred_element_type=jnp.float32)
    return jax.nn.relu(beta * sw + (1.0 - beta) * support)


def _layer_first(net_ref, h_ref, x0_ref, w_ref, hn_ref, netlp_ref, *, beta, scale):
    hn_ref[...] = _to_fp8(_gcn_core(net_ref, h_ref, x0_ref, w_ref, beta, 1.0))
    netlp_ref[...] = _to_fp8(net_ref[...] * scale)


def _layer_mid(net_ref, h_ref, x0_ref, w_ref, hn_ref, *, beta, descale):
    hn_ref[...] = _to_fp8(
        _gcn_core(net_ref, h_ref, x0_ref, w_ref, beta, descale)
    )


def _layer_last(net_ref, h_ref, x0_ref, w_ref, wc_ref, bc_ref, pred_ref, *, beta,
                descale):
    h = _gcn_core(net_ref, h_ref, x0_ref, w_ref, beta, descale)
    pred_ref[...] = (
        jnp.dot(h, wc_ref[...], preferred_element_type=jnp.float32) + bc_ref[...]
    )


def kernel(x, net, bn_gamma, bn_beta, w_in, b_in, gcn_weights, w_cls, b_cls):
    n, nfeat = x.shape
    dim = w_in.shape[1]
    nclass = w_cls.shape[1]
    br = 400 if n % 400 == 0 else n
    nblk = n // br
    scale = 2.0 ** math.floor(math.log2(447.0 * n))
    descale = 1.0 / scale

    g2 = bn_gamma.reshape(1, nfeat)
    b2 = bn_beta.reshape(1, nfeat)
    bi2 = b_in.reshape(1, dim)
    bc2 = b_cls.reshape(1, nclass)

    x0, h = pl.pallas_call(
        _prelude,
        out_shape=[
            jax.ShapeDtypeStruct((n, dim), jnp.float32),
            jax.ShapeDtypeStruct((n, dim), jnp.bfloat16),
        ],
    )(x, g2, b2, w_in, bi2)

    full = lambda i: (0, 0)
    rows = lambda i: (i, 0)
    h_spec = pl.BlockSpec((n, dim), full)
    x0_spec = pl.BlockSpec((br, dim), rows)
    w_spec = pl.BlockSpec((dim, dim), full)
    hn_spec = pl.BlockSpec((br, dim), rows)
    hn_shape = jax.ShapeDtypeStruct((n, dim), FP8)
    params = pltpu.CompilerParams(dimension_semantics=("parallel",))

    net_lp = None
    for l in range(NLAYERS):
        beta = math.log(LAMDA / (l + 1) + 1.0)
        wl = gcn_weights[l]
        if l == 0:
            h, net_lp = pl.pallas_call(
                functools.partial(_layer_first, beta=beta, scale=scale),
                grid=(nblk,),
                in_specs=[pl.BlockSpec((br, n), rows), h_spec, x0_spec, w_spec],
                out_specs=[hn_spec, pl.BlockSpec((br, n), rows)],
                out_shape=[hn_shape, jax.ShapeDtypeStruct((n, n), FP8)],
                compiler_params=params,
            )(net, h, x0, wl)
        elif l < NLAYERS - 1:
            h = pl.pallas_call(
                functools.partial(_layer_mid, beta=beta, descale=descale),
                grid=(nblk,),
                in_specs=[pl.BlockSpec((br, n), rows), h_spec, x0_spec, w_spec],
                out_specs=hn_spec,
                out_shape=hn_shape,
                compiler_params=params,
            )(net_lp, h, x0, wl)
        else:
            pred = pl.pallas_call(
                functools.partial(_layer_last, beta=beta, descale=descale),
                grid=(nblk,),
                in_specs=[
                    pl.BlockSpec((br, n), rows), h_spec, x0_spec, w_spec,
                    pl.BlockSpec((dim, nclass), full),
                    pl.BlockSpec((1, nclass), full),
                ],
                out_specs=pl.BlockSpec((br, nclass), rows),
                out_shape=jax.ShapeDtypeStruct((n, nclass), jnp.float32),
                compiler_params=params,
            )(net_lp, h, x0, wl, w_cls, bc2)
    return pred


# speedup vs baseline: 2.1443x; 1.0951x over previous
import functools
import math

import jax
import jax.numpy as jnp
from jax.experimental import pallas as pl
from jax.experimental.pallas import tpu as pltpu

ALPHA = 0.5
LAMDA = 0.5
NLAYERS = 8
FP8 = jnp.float8_e4m3fn


def _prelude(x_ref, g_ref, b_ref, w_ref, bi_ref, x0_ref, h0_ref):
    x = x_ref[...]
    mean = jnp.mean(x, axis=0, keepdims=True)
    var = jnp.mean((x - mean) ** 2, axis=0, keepdims=True)
    xn = (x - mean) / jnp.sqrt(var + 1e-5) * g_ref[...] + b_ref[...]
    x0 = jax.nn.relu(
        jnp.dot(xn, w_ref[...], preferred_element_type=jnp.float32) + bi_ref[...]
    )
    x0_ref[...] = x0
    h0_ref[...] = x0.astype(jnp.bfloat16)


def _to_fp8(h):
    return jnp.minimum(h, 448.0).astype(FP8)


def _gcn_core(net_ref, h_ref, x0_ref, w_ref, beta, descale):
    if net_ref.dtype == jnp.float32:
        hi = jnp.dot(
            net_ref[...].astype(jnp.bfloat16), h_ref[...],
            preferred_element_type=jnp.float32,
        )
    else:
        hi = jnp.dot(net_ref[...], h_ref[...], preferred_element_type=jnp.float32)
    support = (1.0 - ALPHA) * descale * hi + ALPHA * x0_ref[...]
    sw = jnp.dot(support, w_ref[...], preferred_element_type=jnp.float32)
    return jax.nn.relu(beta * sw + (1.0 - beta) * support)


def _layer_first(net_ref, h_ref, x0_ref, w_ref, hn_ref, netlp_ref, *, beta, scale):
    hn_ref[...] = _to_fp8(_gcn_core(net_ref, h_ref, x0_ref, w_ref, beta, 1.0))
    netlp_ref[...] = _to_fp8(net_ref[...] * scale)


def _layer_mid(net_ref, h_ref, x0_ref, w_ref, hn_ref, *, beta, descale):
    hn_ref[...] = _to_fp8(
        _gcn_core(net_ref, h_ref, x0_ref, w_ref, beta, descale)
    )


def _layer_last(net_ref, h_ref, x0_ref, w_ref, wc_ref, bc_ref, pred_ref, *, beta,
                descale):
    h = _gcn_core(net_ref, h_ref, x0_ref, w_ref, beta, descale)
    pred_ref[...] = (
        jnp.dot(h, wc_ref[...], preferred_element_type=jnp.float32) + bc_ref[...]
    )


def kernel(x, net, bn_gamma, bn_beta, w_in, b_in, gcn_weights, w_cls, b_cls):
    n, nfeat = x.shape
    dim = w_in.shape[1]
    nclass = w_cls.shape[1]
    br = 400 if n % 400 == 0 else n
    nblk = n // br
    brm = 1000 if n % 1000 == 0 else br
    nblkm = n // brm
    scale = 2.0 ** math.floor(math.log2(447.0 * n))
    descale = 1.0 / scale

    g2 = bn_gamma.reshape(1, nfeat)
    b2 = bn_beta.reshape(1, nfeat)
    bi2 = b_in.reshape(1, dim)
    bc2 = b_cls.reshape(1, nclass)

    x0, h = pl.pallas_call(
        _prelude,
        out_shape=[
            jax.ShapeDtypeStruct((n, dim), jnp.float32),
            jax.ShapeDtypeStruct((n, dim), jnp.bfloat16),
        ],
    )(x, g2, b2, w_in, bi2)

    full = lambda i: (0, 0)
    rows = lambda i: (i, 0)
    h_spec = pl.BlockSpec((n, dim), full)
    x0_spec = pl.BlockSpec((br, dim), rows)
    w_spec = pl.BlockSpec((dim, dim), full)
    hn_spec = pl.BlockSpec((br, dim), rows)
    hn_shape = jax.ShapeDtypeStruct((n, dim), FP8)
    params = pltpu.CompilerParams(dimension_semantics=("parallel",))

    net_lp = None
    for l in range(NLAYERS):
        beta = math.log(LAMDA / (l + 1) + 1.0)
        wl = gcn_weights[l]
        if l == 0:
            h, net_lp = pl.pallas_call(
                functools.partial(_layer_first, beta=beta, scale=scale),
                grid=(nblk,),
                in_specs=[pl.BlockSpec((br, n), rows), h_spec, x0_spec, w_spec],
                out_specs=[hn_spec, pl.BlockSpec((br, n), rows)],
                out_shape=[hn_shape, jax.ShapeDtypeStruct((n, n), FP8)],
                compiler_params=params,
            )(net, h, x0, wl)
        elif l < NLAYERS - 1:
            h = pl.pallas_call(
                functools.partial(_layer_mid, beta=beta, descale=descale),
                grid=(nblkm,),
                in_specs=[pl.BlockSpec((brm, n), rows), h_spec,
                          pl.BlockSpec((brm, dim), rows), w_spec],
                out_specs=pl.BlockSpec((brm, dim), rows),
                out_shape=hn_shape,
                compiler_params=params,
            )(net_lp, h, x0, wl)
        else:
            pred = pl.pallas_call(
                functools.partial(_layer_last, beta=beta, descale=descale),
                grid=(nblkm,),
                in_specs=[
                    pl.BlockSpec((brm, n), rows), h_spec,
                    pl.BlockSpec((brm, dim), rows), w_spec,
                    pl.BlockSpec((dim, nclass), full),
                    pl.BlockSpec((1, nclass), full),
                ],
                out_specs=pl.BlockSpec((brm, nclass), rows),
                out_shape=jax.ShapeDtypeStruct((n, nclass), jnp.float32),
                compiler_params=params,
            )(net_lp, h, x0, wl, w_cls, bc2)
    return pred


# speedup vs baseline: 2.3111x; 1.0778x over previous
import functools
import math

import jax
import jax.numpy as jnp
from jax.experimental import pallas as pl
from jax.experimental.pallas import tpu as pltpu

ALPHA = 0.5
LAMDA = 0.5
NLAYERS = 8
FP8 = jnp.float8_e4m3fn


def _prelude(x_ref, g_ref, b_ref, w_ref, bi_ref, x0_ref, h0_ref):
    x = x_ref[...]
    mean = jnp.mean(x, axis=0, keepdims=True)
    var = jnp.mean((x - mean) ** 2, axis=0, keepdims=True)
    xn = (x - mean) / jnp.sqrt(var + 1e-5) * g_ref[...] + b_ref[...]
    x0 = jax.nn.relu(
        jnp.dot(xn, w_ref[...], preferred_element_type=jnp.float32) + bi_ref[...]
    )
    x0_ref[...] = x0
    h0_ref[...] = x0.astype(jnp.bfloat16)


def _to_fp8(h):
    return jnp.minimum(h, 448.0).astype(FP8)


def _layer_first(net_ref, h_ref, x0_ref, w_ref, hn_ref, netlp_ref, *, scale):
    hi = jnp.dot(
        net_ref[...].astype(jnp.bfloat16), h_ref[...],
        preferred_element_type=jnp.float32,
    )
    support = (1.0 - ALPHA) * hi + ALPHA * x0_ref[...]
    hn = jax.nn.relu(
        jnp.dot(support, w_ref[...], preferred_element_type=jnp.float32)
    )
    hn_ref[...] = _to_fp8(hn)
    netlp_ref[...] = _to_fp8(net_ref[...] * scale)


def _mega(netlp_ref, h1_ref, x0_ref, w_ref, wc_ref, bc_ref, pred_ref, hbuf_ref,
          *, nlm, brm, descale):
    l = pl.program_id(0)
    i = pl.program_id(1)

    @pl.when((l == 0) & (i == 0))
    def _():
        hbuf_ref[0] = h1_ref[...]

    cur = l % 2
    h = hbuf_ref[cur]
    hi = jnp.dot(netlp_ref[...], h, preferred_element_type=jnp.float32)
    support = (1.0 - ALPHA) * descale * hi + ALPHA * x0_ref[pl.ds(i * brm, brm), :]
    hn = jax.nn.relu(
        jnp.dot(support, w_ref[0], preferred_element_type=jnp.float32)
    )

    @pl.when(l < nlm - 1)
    def _():
        hbuf_ref[1 - cur, pl.ds(i * brm, brm), :] = _to_fp8(hn)

    @pl.when(l == nlm - 1)
    def _():
        pred_ref[...] = (
            jnp.dot(hn, wc_ref[...], preferred_element_type=jnp.float32)
            + bc_ref[...]
        )


def kernel(x, net, bn_gamma, bn_beta, w_in, b_in, gcn_weights, w_cls, b_cls):
    n, nfeat = x.shape
    dim = w_in.shape[1]
    nclass = w_cls.shape[1]
    br = 400 if n % 400 == 0 else n
    nblk = n // br
    brm = 1000 if n % 1000 == 0 else br
    nblkm = n // brm
    nlm = NLAYERS - 1
    scale = 2.0 ** math.floor(math.log2(447.0 * n))
    descale = 1.0 / scale

    g2 = bn_gamma.reshape(1, nfeat)
    b2 = bn_beta.reshape(1, nfeat)
    bi2 = b_in.reshape(1, dim)
    bc2 = b_cls.reshape(1, nclass)

    betas = jnp.array(
        [math.log(LAMDA / (l + 1) + 1.0) for l in range(NLAYERS)],
        dtype=jnp.float32,
    )
    eye = jnp.eye(dim, dtype=jnp.float32)
    w_mod = betas[:, None, None] * gcn_weights + (1.0 - betas)[:, None, None] * eye

    x0, h = pl.pallas_call(
        _prelude,
        out_shape=[
            jax.ShapeDtypeStruct((n, dim), jnp.float32),
            jax.ShapeDtypeStruct((n, dim), jnp.bfloat16),
        ],
    )(x, g2, b2, w_in, bi2)

    full = lambda i: (0, 0)
    rows = lambda i: (i, 0)

    h, net_lp = pl.pallas_call(
        functools.partial(_layer_first, scale=scale),
        grid=(nblk,),
        in_specs=[
            pl.BlockSpec((br, n), rows),
            pl.BlockSpec((n, dim), full),
            pl.BlockSpec((br, dim), rows),
            pl.BlockSpec((dim, dim), full),
        ],
        out_specs=[pl.BlockSpec((br, dim), rows), pl.BlockSpec((br, n), rows)],
        out_shape=[
            jax.ShapeDtypeStruct((n, dim), FP8),
            jax.ShapeDtypeStruct((n, n), FP8),
        ],
        compiler_params=pltpu.CompilerParams(
            dimension_semantics=("arbitrary",)
        ),
    )(net, h, x0, w_mod[0])

    pred = pl.pallas_call(
        functools.partial(_mega, nlm=nlm, brm=brm, descale=descale),
        grid=(nlm, nblkm),
        in_specs=[
            pl.BlockSpec((brm, n), lambda l, i: (i, 0)),
            pl.BlockSpec((n, dim), lambda l, i: (0, 0)),
            pl.BlockSpec((n, dim), lambda l, i: (0, 0)),
            pl.BlockSpec((1, dim, dim), lambda l, i: (l, 0, 0)),
            pl.BlockSpec((dim, nclass), lambda l, i: (0, 0)),
            pl.BlockSpec((1, nclass), lambda l, i: (0, 0)),
        ],
        out_specs=pl.BlockSpec((brm, nclass), lambda l, i: (i, 0)),
        out_shape=jax.ShapeDtypeStruct((n, nclass), jnp.float32),
        scratch_shapes=[pltpu.VMEM((2, n, dim), FP8)],
        compiler_params=pltpu.CompilerParams(
            dimension_semantics=("arbitrary", "arbitrary")
        ),
    )(net_lp, h, x0, w_mod[1:], w_cls, bc2)
    return pred


# speedup vs baseline: 2.4001x; 1.0385x over previous
import functools
import math

import jax
import jax.numpy as jnp
from jax.experimental import pallas as pl
from jax.experimental.pallas import tpu as pltpu

ALPHA = 0.5
LAMDA = 0.5
NLAYERS = 8
FP8 = jnp.float8_e4m3fn


def _prelude(x_ref, g_ref, b_ref, w_ref, bi_ref, x0_ref, h0_ref):
    x = x_ref[...]
    mean = jnp.mean(x, axis=0, keepdims=True)
    var = jnp.mean((x - mean) ** 2, axis=0, keepdims=True)
    xn = (x - mean) / jnp.sqrt(var + 1e-5) * g_ref[...] + b_ref[...]
    x0 = jax.nn.relu(
        jnp.dot(xn, w_ref[...], preferred_element_type=jnp.float32) + bi_ref[...]
    )
    x0_ref[...] = x0
    h0_ref[...] = x0.astype(jnp.bfloat16)


def _to_fp8(h):
    return jnp.minimum(h, 448.0).astype(FP8)


def _layer_first(net_ref, h_ref, x0_ref, w_ref, hn_ref, netlpa_ref, netlpb_ref,
                 *, scale, nh):
    hi = jnp.dot(
        net_ref[...].astype(jnp.bfloat16), h_ref[...],
        preferred_element_type=jnp.float32,
    )
    support = (1.0 - ALPHA) * hi + ALPHA * x0_ref[...]
    hn = jax.nn.relu(
        jnp.dot(support, w_ref[...], preferred_element_type=jnp.float32)
    )
    hn_ref[...] = _to_fp8(hn)
    netlpa_ref[...] = _to_fp8(net_ref[:, :nh] * scale)
    netlpb_ref[...] = _to_fp8(net_ref[:, nh:] * scale)


def _mega(neta_ref, netb_ref, h1_ref, x0_ref, w_ref, wc_ref, bc_ref, pred_ref,
          hbuf_ref, *, nlm, brm, nh, descale):
    l = pl.program_id(0)
    i = pl.program_id(1)

    @pl.when((l == 0) & (i == 0))
    def _():
        hbuf_ref[0] = h1_ref[...]

    cur = l % 2
    hi = jnp.dot(
        neta_ref[...], hbuf_ref[cur, :nh, :], preferred_element_type=jnp.float32
    ) + jnp.dot(
        netb_ref[...], hbuf_ref[cur, nh:, :], preferred_element_type=jnp.float32
    )
    x0b = x0_ref[pl.ds(i * brm, brm), :].astype(jnp.float32)
    support = (1.0 - ALPHA) * descale * hi + ALPHA * x0b
    hn = jax.nn.relu(
        jnp.dot(support, w_ref[0], preferred_element_type=jnp.float32)
    )

    @pl.when(l < nlm - 1)
    def _():
        hbuf_ref[1 - cur, pl.ds(i * brm, brm), :] = _to_fp8(hn)

    @pl.when(l == nlm - 1)
    def _():
        pred_ref[...] = (
            jnp.dot(hn, wc_ref[...], preferred_element_type=jnp.float32)
            + bc_ref[...]
        )


def kernel(x, net, bn_gamma, bn_beta, w_in, b_in, gcn_weights, w_cls, b_cls):
    n, nfeat = x.shape
    dim = w_in.shape[1]
    nclass = w_cls.shape[1]
    br = 400 if n % 400 == 0 else n
    nblk = n // br
    brm = 1000 if n % 1000 == 0 else br
    nblkm = n // brm
    nlm = NLAYERS - 1
    scale = 2.0 ** math.floor(math.log2(447.0 * n))
    descale = 1.0 / scale

    g2 = bn_gamma.reshape(1, nfeat)
    b2 = bn_beta.reshape(1, nfeat)
    bi2 = b_in.reshape(1, dim)
    bc2 = b_cls.reshape(1, nclass)

    betas = jnp.array(
        [math.log(LAMDA / (l + 1) + 1.0) for l in range(NLAYERS)],
        dtype=jnp.float32,
    )
    eye = jnp.eye(dim, dtype=jnp.float32)
    w_mod = betas[:, None, None] * gcn_weights + (1.0 - betas)[:, None, None] * eye

    x0, x0b16 = pl.pallas_call(
        _prelude,
        out_shape=[
            jax.ShapeDtypeStruct((n, dim), jnp.float32),
            jax.ShapeDtypeStruct((n, dim), jnp.bfloat16),
        ],
    )(x, g2, b2, w_in, bi2)
    h = x0b16

    full = lambda i: (0, 0)
    rows = lambda i: (i, 0)

    nh = n // 2
    h, net_lpa, net_lpb = pl.pallas_call(
        functools.partial(_layer_first, scale=scale, nh=nh),
        grid=(nblk,),
        in_specs=[
            pl.BlockSpec((br, n), rows),
            pl.BlockSpec((n, dim), full),
            pl.BlockSpec((br, dim), rows),
            pl.BlockSpec((dim, dim), full),
        ],
        out_specs=[
            pl.BlockSpec((br, dim), rows),
            pl.BlockSpec((br, nh), rows),
            pl.BlockSpec((br, nh), rows),
        ],
        out_shape=[
            jax.ShapeDtypeStruct((n, dim), FP8),
            jax.ShapeDtypeStruct((n, nh), FP8),
            jax.ShapeDtypeStruct((n, nh), FP8),
        ],
        compiler_params=pltpu.CompilerParams(
            dimension_semantics=("arbitrary",)
        ),
    )(net, h, x0, w_mod[0])

    pred = pl.pallas_call(
        functools.partial(_mega, nlm=nlm, brm=brm, nh=nh, descale=descale),
        grid=(nlm, nblkm),
        in_specs=[
            pl.BlockSpec((brm, nh), lambda l, i: (i, 0)),
            pl.BlockSpec((brm, nh), lambda l, i: (i, 0)),
            pl.BlockSpec((n, dim), lambda l, i: (0, 0)),
            pl.BlockSpec((n, dim), lambda l, i: (0, 0)),
            pl.BlockSpec((1, dim, dim), lambda l, i: (l, 0, 0)),
            pl.BlockSpec((dim, nclass), lambda l, i: (0, 0)),
            pl.BlockSpec((1, nclass), lambda l, i: (0, 0)),
        ],
        out_specs=pl.BlockSpec((brm, nclass), lambda l, i: (i, 0)),
        out_shape=jax.ShapeDtypeStruct((n, nclass), jnp.float32),
        scratch_shapes=[pltpu.VMEM((2, n, dim), FP8)],
        compiler_params=pltpu.CompilerParams(
            dimension_semantics=("arbitrary", "arbitrary")
        ),
    )(net_lpa, net_lpb, h, x0b16, w_mod[1:], w_cls, bc2)
    return pred
